# baseline (speedup 1.0000x reference)
import functools

import jax
import jax.numpy as jnp
from jax import lax
from jax.experimental import pallas as pl
from jax.experimental.pallas import tpu as pltpu
from jax.experimental.pallas import tpu_sc as plsc

_NVEC = (6, 4, 4, 4, 4, 7, 49)
_OFFS = (0, 6, 10, 14, 18, 22, 29, 78)
_TOTAL = 78
_NP = 7
_MAPSIZE = 256
_B = 1024
_MASK_VALUE = -1e8

_BTC = 512
_BSC = _B - _BTC

_NC = 2
_NS = 16
_NW = _NC * _NS
_GROUPS = _MAPSIZE // 16
_WORDS_PER_ENV = _MAPSIZE * _TOTAL // 4
_LN2 = 0.6931471805599453

_ENVS_PER_BLOCK = 32
_ROWS_PER_BLOCK = _ENVS_PER_BLOCK * _MAPSIZE


def _ln16(z):
    b = plsc.bitcast(z, jnp.int32)
    ex = lax.shift_right_logical(b, 23) - 127
    m = plsc.bitcast((b & 0x7FFFFF) | (127 << 23), jnp.float32)
    y = (m - 1.0) / (m + 1.0)
    y2 = y * y
    p = y2 * (1.0 / 7.0 + y2 * (1.0 / 9.0))
    p = 2.0 * y * (1.0 + y2 * (1.0 / 3.0 + y2 * (0.2 + p)))
    return p + ex.astype(jnp.float32) * _LN2


def _make_sc_kernel(bsc, env0):
    env_per_w = bsc // _NW

    def body(x_hbm, mw_hbm, a_hbm, lp_hbm, ent_hbm,
             xbuf, mbuf, abuf, lpbuf, entbuf):
        wid = lax.axis_index("s") * _NC + lax.axis_index("c")
        lanes = lax.iota(jnp.int32, 16)
        zero16 = jnp.zeros((16,), jnp.float32)

        def env_body(e, carry):
            env = env0 + wid * env_per_w + e
            pltpu.sync_copy(x_hbm.at[env], xbuf)
            pltpu.sync_copy(mw_hbm.at[env], mbuf)
            pltpu.sync_copy(a_hbm.at[env], abuf)

            def group_body(g, acc):
                acc_lp, acc_ent = acc
                rows = g * 16 + lanes
                rb = rows * _TOTAL
                Zs = [zero16 for _ in range(_NP)]
                Ws = [zero16 for _ in range(_NP)]
                for i in range(_NP):
                    Zi = Zs[i]
                    Wi = Ws[i]
                    for t in range(_OFFS[i], _OFFS[i + 1]):
                        bidx = rb + t
                        xv = plsc.load_gather(xbuf, [bidx])
                        word = lax.shift_right_logical(bidx, 2)
                        sh = (bidx & 3) * 8
                        mword = plsc.load_gather(mbuf, [word])
                        bit = lax.shift_right_logical(mword, sh) & 1
                        ev = jnp.exp(xv) * bit.astype(jnp.float32)
                        Zi = Zi + ev
                        Wi = Wi + xv * ev
                    Zs[i] = Zi
                    Ws[i] = Wi
                for i in range(_NP):
                    ai = plsc.load_gather(abuf, [rows * _NP + i])
                    col = ai + _OFFS[i]
                    bidx = rb + col
                    xa = plsc.load_gather(xbuf, [bidx])
                    word = lax.shift_right_logical(bidx, 2)
                    sh = (bidx & 3) * 8
                    mword = plsc.load_gather(mbuf, [word])
                    bit = lax.shift_right_logical(mword, sh) & 1
                    mxa = jnp.where(bit == 1, xa, _MASK_VALUE)
                    lz = _ln16(Zs[i])
                    acc_lp = acc_lp + (mxa - lz)
                    acc_ent = acc_ent + (lz - Ws[i] / Zs[i])
                return acc_lp, acc_ent

            acc_lp, acc_ent = lax.fori_loop(0, _GROUPS, group_body,
                                            (zero16, zero16))
            ev16 = jnp.broadcast_to(e, (16,)).astype(jnp.int32)
            lane0 = lanes == 0
            plsc.store_scatter(lpbuf, [ev16],
                               jnp.broadcast_to(jnp.sum(acc_lp), (16,)),
                               mask=lane0)
            plsc.store_scatter(entbuf, [ev16],
                               jnp.broadcast_to(jnp.sum(acc_ent), (16,)),
                               mask=lane0)
            return carry

        lax.fori_loop(0, env_per_w, env_body, 0)
        pltpu.sync_copy(lpbuf, lp_hbm.at[wid])
        pltpu.sync_copy(entbuf, ent_hbm.at[wid])

    return pl.kernel(
        body,
        out_type=[jax.ShapeDtypeStruct((_NW, bsc // _NW), jnp.float32),
                  jax.ShapeDtypeStruct((_NW, bsc // _NW), jnp.float32)],
        mesh=plsc.VectorSubcoreMesh(core_axis_name="c", subcore_axis_name="s"),
        compiler_params=pltpu.CompilerParams(needs_layout_passes=False),
        scratch_types=[
            pltpu.VMEM((_MAPSIZE * _TOTAL,), jnp.float32),
            pltpu.VMEM((_WORDS_PER_ENV,), jnp.int32),
            pltpu.VMEM((_MAPSIZE * _NP,), jnp.int32),
            pltpu.VMEM((bsc // _NW,), jnp.float32),
            pltpu.VMEM((bsc // _NW,), jnp.float32),
        ],
    )


def _seg_matrix():
    it = lax.broadcasted_iota(jnp.int32, (_TOTAL, 1), 0)
    cols = [((it >= _OFFS[i]) & (it < _OFFS[i + 1])).astype(jnp.float32)
            for i in range(_NP)]
    return jnp.concatenate(cols, axis=1)


def _seg_offsets_row():
    it = lax.broadcasted_iota(jnp.int32, (1, _TOTAL), 1)
    r = jnp.zeros((1, _TOTAL), jnp.float32)
    for i in range(1, _NP):
        r = jnp.where(it >= _OFFS[i], float(_OFFS[i]), r)
    return r


def _tc_body(x_ref, m_ref, a_ref, lp_ref, ent_ref):
    R = _ROWS_PER_BLOCK
    x = x_ref[...].reshape(R, _TOTAL)
    msk = m_ref[...].reshape(R, _TOTAL)
    S = _seg_matrix()
    mx = jnp.where(msk, x, _MASK_VALUE)
    e = jnp.exp(mx)
    we = mx * e
    Z = lax.dot(e, S)
    W = lax.dot(we, S)
    logZ = jnp.log(Z)

    act = a_ref[...].reshape(R, _NP).astype(jnp.float32)
    dn_t = (((1,), (1,)), ((), ()))
    tgt = lax.dot_general(act, S, dn_t) + _seg_offsets_row()
    iota = lax.broadcasted_iota(jnp.int32, (R, _TOTAL), 1).astype(jnp.float32)
    g_all = jnp.sum(jnp.where(iota == tgt, mx, 0.0), -1, keepdims=True)

    lp_row = g_all - jnp.sum(logZ, -1, keepdims=True)
    ent_row = jnp.sum(logZ - W / Z, -1, keepdims=True)

    ne = _ENVS_PER_BLOCK
    row_env = lax.broadcasted_iota(jnp.int32, (R, ne), 0) // _MAPSIZE
    env_id = lax.broadcasted_iota(jnp.int32, (R, ne), 1)
    sel = (row_env == env_id).astype(jnp.float32)
    dn = (((0,), (0,)), ((), ()))
    lp_ref[...] = lax.dot_general(lp_row, sel, dn)[None]
    ent_ref[...] = lax.dot_general(ent_row, sel, dn)[None]


def _tc_part(x_logits, invalid_action_masks, action, btc):
    ne = _ENVS_PER_BLOCK
    nblocks = btc // ne
    lp, ent = pl.pallas_call(
        _tc_body,
        grid=(nblocks,),
        in_specs=[
            pl.BlockSpec((ne, _MAPSIZE, _TOTAL), lambda i: (i, 0, 0)),
            pl.BlockSpec((ne, _MAPSIZE, _TOTAL), lambda i: (i, 0, 0)),
            pl.BlockSpec((ne, _MAPSIZE, _NP), lambda i: (i, 0, 0)),
        ],
        out_specs=[
            pl.BlockSpec((1, 1, ne), lambda i: (i, 0, 0)),
            pl.BlockSpec((1, 1, ne), lambda i: (i, 0, 0)),
        ],
        out_shape=[
            jax.ShapeDtypeStruct((nblocks, 1, ne), jnp.float32),
            jax.ShapeDtypeStruct((nblocks, 1, ne), jnp.float32),
        ],
    )(x_logits, invalid_action_masks, action)
    return lp.reshape(btc), ent.reshape(btc)


@jax.jit
def kernel(x_logits, invalid_action_masks, action):
    x_sc = x_logits.reshape(_B, _MAPSIZE * _TOTAL)
    m_sc = invalid_action_masks.view(jnp.uint8)
    mw_sc = lax.bitcast_convert_type(
        m_sc.reshape(_B, _WORDS_PER_ENV, 4), jnp.int32)
    a_sc = action.reshape(_B, _MAPSIZE * _NP)
    lp_sc, ent_sc = _make_sc_kernel(_BSC, _BTC)(x_sc, mw_sc, a_sc)
    lp_sc = lp_sc.reshape(_BSC)
    ent_sc = ent_sc.reshape(_BSC)

    lp_tc, ent_tc = _tc_part(x_logits, invalid_action_masks, action, _BTC)

    lp = jnp.concatenate([lp_tc, lp_sc])
    ent = jnp.concatenate([ent_tc, ent_sc])
    return action, lp, ent


# speedup vs baseline: 1.4293x; 1.4293x over previous
import functools

import jax
import jax.numpy as jnp
from jax import lax
from jax.experimental import pallas as pl
from jax.experimental.pallas import tpu as pltpu
from jax.experimental.pallas import tpu_sc as plsc

_NVEC = (6, 4, 4, 4, 4, 7, 49)
_OFFS = (0, 6, 10, 14, 18, 22, 29, 78)
_TOTAL = 78
_NP = 7
_MAPSIZE = 256
_B = 1024
_MASK_VALUE = -1e8

_BTC = 0
_BSC = _B - _BTC

_NC = 2
_NS = 16
_NW = _NC * _NS
_GROUPS = _MAPSIZE // 16
_WORDS_PER_ENV = _MAPSIZE * _TOTAL // 4
_LN2 = 0.6931471805599453

_ENVS_PER_BLOCK = 32
_ROWS_PER_BLOCK = _ENVS_PER_BLOCK * _MAPSIZE


def _ln16(z):
    b = plsc.bitcast(z, jnp.int32)
    ex = lax.shift_right_logical(b, 23) - 127
    m = plsc.bitcast((b & 0x7FFFFF) | (127 << 23), jnp.float32)
    y = (m - 1.0) / (m + 1.0)
    y2 = y * y
    p = y2 * (1.0 / 7.0 + y2 * (1.0 / 9.0))
    p = 2.0 * y * (1.0 + y2 * (1.0 / 3.0 + y2 * (0.2 + p)))
    return p + ex.astype(jnp.float32) * _LN2


def _make_sc_kernel(bsc, env0):
    env_per_w = bsc // _NW

    def body(x_hbm, mw_hbm, a_hbm, lp_hbm, ent_hbm,
             xbuf0, mbuf0, abuf0, xbuf1, mbuf1, abuf1,
             lpbuf, entbuf, sem0, sem1):
        wid = lax.axis_index("s") * _NC + lax.axis_index("c")
        lanes = lax.iota(jnp.int32, 16)
        zero16 = jnp.zeros((16,), jnp.float32)
        bufs = ((xbuf0, mbuf0, abuf0, sem0), (xbuf1, mbuf1, abuf1, sem1))
        wbase = env0 + wid * env_per_w

        def start(env, b):
            xb, mb, ab, sem = bufs[b]
            pltpu.async_copy(x_hbm.at[env], xb, sem)
            pltpu.async_copy(mw_hbm.at[env], mb, sem)
            pltpu.async_copy(a_hbm.at[env], ab, sem)

        def wait(env, b):
            xb, mb, ab, sem = bufs[b]
            pltpu.make_async_copy(x_hbm.at[env], xb, sem).wait()
            pltpu.make_async_copy(mw_hbm.at[env], mb, sem).wait()
            pltpu.make_async_copy(a_hbm.at[env], ab, sem).wait()

        def process(xbuf, mbuf, abuf, e):
            def group_body(g, acc):
                acc_lp, acc_ent = acc
                rows = g * 16 + lanes
                rb = rows * _TOTAL
                Zs = [zero16 for _ in range(_NP)]
                Ws = [zero16 for _ in range(_NP)]
                for i in range(_NP):
                    Zi = Zs[i]
                    Wi = Ws[i]
                    for t in range(_OFFS[i], _OFFS[i + 1]):
                        bidx = rb + t
                        xv = plsc.load_gather(xbuf, [bidx])
                        word = lax.shift_right_logical(bidx, 2)
                        sh = (bidx & 3) * 8
                        mword = plsc.load_gather(mbuf, [word])
                        bit = lax.shift_right_logical(mword, sh) & 1
                        ev = jnp.exp(xv) * bit.astype(jnp.float32)
                        Zi = Zi + ev
                        Wi = Wi + xv * ev
                    Zs[i] = Zi
                    Ws[i] = Wi
                for i in range(_NP):
                    ai = plsc.load_gather(abuf, [rows * _NP + i])
                    col = ai + _OFFS[i]
                    bidx = rb + col
                    xa = plsc.load_gather(xbuf, [bidx])
                    word = lax.shift_right_logical(bidx, 2)
                    sh = (bidx & 3) * 8
                    mword = plsc.load_gather(mbuf, [word])
                    bit = lax.shift_right_logical(mword, sh) & 1
                    mxa = jnp.where(bit == 1, xa, _MASK_VALUE)
                    lz = _ln16(Zs[i])
                    acc_lp = acc_lp + (mxa - lz)
                    acc_ent = acc_ent + (lz - Ws[i] / Zs[i])
                return acc_lp, acc_ent

            acc_lp, acc_ent = lax.fori_loop(0, _GROUPS, group_body,
                                            (zero16, zero16))
            ev16 = jnp.broadcast_to(e, (16,)).astype(jnp.int32)
            lane0 = lanes == 0
            plsc.store_scatter(lpbuf, [ev16],
                               jnp.broadcast_to(jnp.sum(acc_lp), (16,)),
                               mask=lane0)
            plsc.store_scatter(entbuf, [ev16],
                               jnp.broadcast_to(jnp.sum(acc_ent), (16,)),
                               mask=lane0)

        start(wbase, 0)

        def pair_body(p, carry):
            for b in range(2):
                e = 2 * p + b
                env = wbase + e
                wait(env, b)
                nxt = e + 1

                @pl.when(nxt < env_per_w)
                def _():
                    start(wbase + nxt, 1 - b)

                xb, mb, ab, _sem = bufs[b]
                process(xb, mb, ab, e)
            return carry

        lax.fori_loop(0, env_per_w // 2, pair_body, 0)
        pltpu.sync_copy(lpbuf, lp_hbm.at[wid])
        pltpu.sync_copy(entbuf, ent_hbm.at[wid])

    return pl.kernel(
        body,
        out_type=[jax.ShapeDtypeStruct((_NW, bsc // _NW), jnp.float32),
                  jax.ShapeDtypeStruct((_NW, bsc // _NW), jnp.float32)],
        mesh=plsc.VectorSubcoreMesh(core_axis_name="c", subcore_axis_name="s"),
        compiler_params=pltpu.CompilerParams(needs_layout_passes=False),
        scratch_types=[
            pltpu.VMEM((_MAPSIZE * _TOTAL,), jnp.float32),
            pltpu.VMEM((_WORDS_PER_ENV,), jnp.int32),
            pltpu.VMEM((_MAPSIZE * _NP,), jnp.int32),
            pltpu.VMEM((_MAPSIZE * _TOTAL,), jnp.float32),
            pltpu.VMEM((_WORDS_PER_ENV,), jnp.int32),
            pltpu.VMEM((_MAPSIZE * _NP,), jnp.int32),
            pltpu.VMEM((bsc // _NW,), jnp.float32),
            pltpu.VMEM((bsc // _NW,), jnp.float32),
            pltpu.SemaphoreType.DMA,
            pltpu.SemaphoreType.DMA,
        ],
    )


def _seg_matrix():
    it = lax.broadcasted_iota(jnp.int32, (_TOTAL, 1), 0)
    cols = [((it >= _OFFS[i]) & (it < _OFFS[i + 1])).astype(jnp.float32)
            for i in range(_NP)]
    return jnp.concatenate(cols, axis=1)


def _seg_offsets_row():
    it = lax.broadcasted_iota(jnp.int32, (1, _TOTAL), 1)
    r = jnp.zeros((1, _TOTAL), jnp.float32)
    for i in range(1, _NP):
        r = jnp.where(it >= _OFFS[i], float(_OFFS[i]), r)
    return r


def _tc_body(x_ref, m_ref, a_ref, lp_ref, ent_ref):
    R = _ROWS_PER_BLOCK
    x = x_ref[...].reshape(R, _TOTAL)
    msk = m_ref[...].reshape(R, _TOTAL)
    S = _seg_matrix()
    mx = jnp.where(msk, x, _MASK_VALUE)
    e = jnp.exp(mx)
    we = mx * e
    Z = lax.dot(e, S)
    W = lax.dot(we, S)
    logZ = jnp.log(Z)

    act = a_ref[...].reshape(R, _NP).astype(jnp.float32)
    dn_t = (((1,), (1,)), ((), ()))
    tgt = lax.dot_general(act, S, dn_t) + _seg_offsets_row()
    iota = lax.broadcasted_iota(jnp.int32, (R, _TOTAL), 1).astype(jnp.float32)
    g_all = jnp.sum(jnp.where(iota == tgt, mx, 0.0), -1, keepdims=True)

    lp_row = g_all - jnp.sum(logZ, -1, keepdims=True)
    ent_row = jnp.sum(logZ - W / Z, -1, keepdims=True)

    ne = _ENVS_PER_BLOCK
    row_env = lax.broadcasted_iota(jnp.int32, (R, ne), 0) // _MAPSIZE
    env_id = lax.broadcasted_iota(jnp.int32, (R, ne), 1)
    sel = (row_env == env_id).astype(jnp.float32)
    dn = (((0,), (0,)), ((), ()))
    lp_ref[...] = lax.dot_general(lp_row, sel, dn)[None]
    ent_ref[...] = lax.dot_general(ent_row, sel, dn)[None]


def _tc_part(x_logits, invalid_action_masks, action, btc):
    ne = _ENVS_PER_BLOCK
    nblocks = btc // ne
    lp, ent = pl.pallas_call(
        _tc_body,
        grid=(nblocks,),
        in_specs=[
            pl.BlockSpec((ne, _MAPSIZE, _TOTAL), lambda i: (i, 0, 0)),
            pl.BlockSpec((ne, _MAPSIZE, _TOTAL), lambda i: (i, 0, 0)),
            pl.BlockSpec((ne, _MAPSIZE, _NP), lambda i: (i, 0, 0)),
        ],
        out_specs=[
            pl.BlockSpec((1, 1, ne), lambda i: (i, 0, 0)),
            pl.BlockSpec((1, 1, ne), lambda i: (i, 0, 0)),
        ],
        out_shape=[
            jax.ShapeDtypeStruct((nblocks, 1, ne), jnp.float32),
            jax.ShapeDtypeStruct((nblocks, 1, ne), jnp.float32),
        ],
    )(x_logits, invalid_action_masks, action)
    return lp.reshape(btc), ent.reshape(btc)


@jax.jit
def kernel(x_logits, invalid_action_masks, action):
    x_sc = x_logits.reshape(_B, _MAPSIZE * _TOTAL)
    m_sc = invalid_action_masks.view(jnp.uint8)
    mw_sc = lax.bitcast_convert_type(
        m_sc.reshape(_B, _WORDS_PER_ENV, 4), jnp.int32)
    a_sc = action.reshape(_B, _MAPSIZE * _NP)
    lp_sc, ent_sc = _make_sc_kernel(_BSC, _BTC)(x_sc, mw_sc, a_sc)
    lp = lp_sc.reshape(_BSC)
    ent = ent_sc.reshape(_BSC)
    return action, lp, ent


# speedup vs baseline: 1.4296x; 1.0002x over previous
import jax
import jax.numpy as jnp
from jax import lax
from jax.experimental import pallas as pl
from jax.experimental.pallas import tpu as pltpu
from jax.experimental.pallas import tpu_sc as plsc

_NVEC = (6, 4, 4, 4, 4, 7, 49)
_OFFS = (0, 6, 10, 14, 18, 22, 29, 78)
_TOTAL = 78
_NP = 7
_MAPSIZE = 256
_B = 1024
_MASK_VALUE = -1e8

_NC = 2
_NS = 16
_NW = _NC * _NS
_GROUPS = _MAPSIZE // 16
_WORDS_PER_ENV = _MAPSIZE * _TOTAL // 4
_LN2 = 0.6931471805599453


def _ln16(z):
    b = plsc.bitcast(z, jnp.int32)
    ex = lax.shift_right_logical(b, 23) - 127
    m = plsc.bitcast((b & 0x7FFFFF) | (127 << 23), jnp.float32)
    y = (m - 1.0) / (m + 1.0)
    y2 = y * y
    p = y2 * (1.0 / 7.0 + y2 * (1.0 / 9.0))
    p = 2.0 * y * (1.0 + y2 * (1.0 / 3.0 + y2 * (0.2 + p)))
    return p + ex.astype(jnp.float32) * _LN2


def _make_sc_kernel(bsc):
    env_per_w = bsc // _NW

    def body(x_hbm, mw_hbm, a_hbm, lp_hbm, ent_hbm,
             xbuf0, mbuf0, abuf0, xbuf1, mbuf1, abuf1,
             lpbuf, entbuf, sem0, sem1):
        wid = lax.axis_index("s") * _NC + lax.axis_index("c")
        lanes = lax.iota(jnp.int32, 16)
        zero16 = jnp.zeros((16,), jnp.float32)
        bufs = ((xbuf0, mbuf0, abuf0, sem0), (xbuf1, mbuf1, abuf1, sem1))
        wbase = wid * env_per_w

        def start(env, b):
            xb, mb, ab, sem = bufs[b]
            pltpu.async_copy(x_hbm.at[env], xb, sem)
            pltpu.async_copy(mw_hbm.at[env], mb, sem)
            pltpu.async_copy(a_hbm.at[env], ab, sem)

        def wait(env, b):
            xb, mb, ab, sem = bufs[b]
            pltpu.make_async_copy(x_hbm.at[env], xb, sem).wait()
            pltpu.make_async_copy(mw_hbm.at[env], mb, sem).wait()
            pltpu.make_async_copy(a_hbm.at[env], ab, sem).wait()

        def process(xbuf, mbuf, abuf, e):
            def group_body(g, acc):
                acc_lp, acc_ent = acc
                rows = g * 16 + lanes
                rb = rows * _TOTAL
                Zs = [zero16 for _ in range(_NP)]
                Ws = [zero16 for _ in range(_NP)]
                for i in range(_NP):
                    Zi = Zs[i]
                    Wi = Ws[i]
                    for t in range(_OFFS[i], _OFFS[i + 1]):
                        bidx = rb + t
                        xv = plsc.load_gather(xbuf, [bidx])
                        word = lax.shift_right_logical(bidx, 2)
                        sh = (bidx & 3) * 8
                        mword = plsc.load_gather(mbuf, [word])
                        bit = lax.shift_right_logical(mword, sh) & 1
                        ev = jnp.exp(xv) * bit.astype(jnp.float32)
                        Zi = Zi + ev
                        Wi = Wi + xv * ev
                    Zs[i] = Zi
                    Ws[i] = Wi
                for i in range(_NP):
                    ai = plsc.load_gather(abuf, [rows * _NP + i])
                    col = ai + _OFFS[i]
                    bidx = rb + col
                    xa = plsc.load_gather(xbuf, [bidx])
                    word = lax.shift_right_logical(bidx, 2)
                    sh = (bidx & 3) * 8
                    mword = plsc.load_gather(mbuf, [word])
                    bit = lax.shift_right_logical(mword, sh) & 1
                    mxa = jnp.where(bit == 1, xa, _MASK_VALUE)
                    lz = _ln16(Zs[i])
                    acc_lp = acc_lp + (mxa - lz)
                    acc_ent = acc_ent + (lz - Ws[i] / Zs[i])
                return acc_lp, acc_ent

            acc_lp, acc_ent = lax.fori_loop(0, _GROUPS, group_body,
                                            (zero16, zero16))
            ev16 = jnp.broadcast_to(e, (16,)).astype(jnp.int32)
            lane0 = lanes == 0
            plsc.store_scatter(lpbuf, [ev16],
                               jnp.broadcast_to(jnp.sum(acc_lp), (16,)),
                               mask=lane0)
            plsc.store_scatter(entbuf, [ev16],
                               jnp.broadcast_to(jnp.sum(acc_ent), (16,)),
                               mask=lane0)

        start(wbase, 0)

        def pair_body(p, carry):
            for b in range(2):
                e = 2 * p + b
                env = wbase + e
                wait(env, b)
                nxt = e + 1

                @pl.when(nxt < env_per_w)
                def _():
                    start(wbase + nxt, 1 - b)

                xb, mb, ab, _sem = bufs[b]
                process(xb, mb, ab, e)
            return carry

        lax.fori_loop(0, env_per_w // 2, pair_body, 0)
        pltpu.sync_copy(lpbuf, lp_hbm.at[wid])
        pltpu.sync_copy(entbuf, ent_hbm.at[wid])

    return pl.kernel(
        body,
        out_type=[jax.ShapeDtypeStruct((_NW, bsc // _NW), jnp.float32),
                  jax.ShapeDtypeStruct((_NW, bsc // _NW), jnp.float32)],
        mesh=plsc.VectorSubcoreMesh(core_axis_name="c", subcore_axis_name="s"),
        compiler_params=pltpu.CompilerParams(needs_layout_passes=False),
        scratch_types=[
            pltpu.VMEM((_MAPSIZE * _TOTAL,), jnp.float32),
            pltpu.VMEM((_WORDS_PER_ENV,), jnp.int32),
            pltpu.VMEM((_MAPSIZE * _NP,), jnp.int32),
            pltpu.VMEM((_MAPSIZE * _TOTAL,), jnp.float32),
            pltpu.VMEM((_WORDS_PER_ENV,), jnp.int32),
            pltpu.VMEM((_MAPSIZE * _NP,), jnp.int32),
            pltpu.VMEM((bsc // _NW,), jnp.float32),
            pltpu.VMEM((bsc // _NW,), jnp.float32),
            pltpu.SemaphoreType.DMA,
            pltpu.SemaphoreType.DMA,
        ],
    )


@jax.jit
def kernel(x_logits, invalid_action_masks, action):
    x_sc = x_logits.reshape(_B, _MAPSIZE * _TOTAL)
    m_sc = invalid_action_masks.view(jnp.uint8)
    mw_sc = lax.bitcast_convert_type(
        m_sc.reshape(_B, _WORDS_PER_ENV, 4), jnp.int32)
    a_sc = action.reshape(_B, _MAPSIZE * _NP)
    lp_sc, ent_sc = _make_sc_kernel(_B)(x_sc, mw_sc, a_sc)
    lp = lp_sc.reshape(_B)
    ent = ent_sc.reshape(_B)
    return action, lp, ent
